# Initial kernel scaffold; baseline (speedup 1.0000x reference)
import jax
import jax.numpy as jnp
from jax.experimental import pallas as pl


def kernel(x, features, pos_emb, ge_c1a_w, ge_c1a_b, ge_bn1_g, ge_bn1_b, ge_c1b_w, ge_c1b_b, ge_c2a_w, ge_c2a_b, ge_bn2_g, ge_bn2_b, ge_c2b_w, ge_c2b_b, blk_ln1_g, blk_ln1_b, blk_qkv_w, blk_qkv_b, blk_fc_w, blk_fc_b, blk_ln2_g, blk_ln2_b, blk_mlp_w1, blk_mlp_b1, blk_mlp_w2, blk_mlp_b2, he_ln_g, he_ln_b, he_w1, he_b1, he_w2, he_b2, hp_ln_g, hp_ln_b, hp_w1, hp_b1, hp_w2, hp_b2):
    raise NotImplementedError("write your pallas kernel here")


# speedup vs baseline: 1.7949x; 1.7949x over previous
import functools

import jax
import jax.numpy as jnp
from jax import lax
from jax.experimental import pallas as pl
from jax.experimental.pallas import tpu as pltpu
from jax.experimental.pallas import tpu_sc as plsc

N_POINTS = 50000
NT = 392
NPAD = NT * 128
NUM_GROUPS = 512
GROUP_SIZE = 32
CB = 8
NBLK = NUM_GROUPS // CB
INF = float("inf")
IBIG = 1 << 30


def _fps_body(xc_ref, cen_ref):
    li = lax.broadcasted_iota(jnp.int32, (1, 128), 1)
    r0 = lax.broadcasted_iota(jnp.int32, (NT, 128), 0)
    l0 = lax.broadcasted_iota(jnp.int32, (NT, 128), 1)
    fid = r0 * 128 + l0
    valid = fid < N_POINTS

    def body(i, carry):
        f, dist = carry
        r = f // 128
        l = f - r * 128
        row0 = xc_ref[0, pl.ds(r, 1), :]
        row1 = xc_ref[1, pl.ds(r, 1), :]
        row2 = xc_ref[2, pl.ds(r, 1), :]
        sel = (li == l).astype(jnp.float32)
        cx = jnp.sum(row0 * sel)
        cy = jnp.sum(row1 * sel)
        cz = jnp.sum(row2 * sel)
        lif = li.astype(jnp.float32)
        rec = (jnp.where(li == 0, cx, 0.0) + jnp.where(li == 1, cy, 0.0)
               + jnp.where(li == 2, cz, 0.0))
        cen_ref[pl.ds(i, 1), :] = rec
        dx = xc_ref[0] - cx
        dy = xc_ref[1] - cy
        dz = xc_ref[2] - cz
        d = dx * dx + dy * dy + dz * dz
        dist = jnp.where(valid, jnp.minimum(dist, d), dist)
        m = jnp.max(dist)
        f_new = jnp.min(jnp.where(dist == m, fid, IBIG))
        return f_new, dist

    dist0 = jnp.where(valid, jnp.float32(1e10), jnp.float32(-1.0))
    lax.fori_loop(0, NUM_GROUPS, body, (jnp.int32(0), dist0))


def _fps(xc):
    return pl.pallas_call(
        _fps_body,
        out_shape=jax.ShapeDtypeStruct((NUM_GROUPS, 128), jnp.float32),
    )(xc)


def _knn_body(xc_ref, xr_ref, cen_ref, topk_ref, near_ref, d_ref, bestv_ref,
              besti_ref):
    b = pl.program_id(0)
    li = lax.broadcasted_iota(jnp.int32, (1, 128), 1)
    ci8 = lax.broadcasted_iota(jnp.int32, (CB, 1), 0)
    it392 = lax.broadcasted_iota(jnp.int32, (CB, NT), 1)
    is8 = lax.broadcasted_iota(jnp.int32, (CB, NT), 0)
    cr8 = lax.broadcasted_iota(jnp.int32, (CB, 128), 0)
    cl8 = lax.broadcasted_iota(jnp.int32, (CB, 128), 1)

    cen8 = cen_ref[...]
    cx = cen_ref[:, 0:1]
    cy = cen_ref[:, 1:2]
    cz = cen_ref[:, 2:3]
    cc = (cx * cx + cy * cy) + cz * cz

    @pl.when(b == 0)
    def _init():
        bestv_ref[...] = jnp.full((NT, 128), INF)
        besti_ref[...] = jnp.zeros((NT, 128), jnp.int32)

    def fill(t, cm):
        x0 = xc_ref[0, pl.ds(t, 1), :]
        x1 = xc_ref[1, pl.ds(t, 1), :]
        x2 = xc_ref[2, pl.ds(t, 1), :]
        xx = (x0 * x0 + x1 * x1) + x2 * x2
        x_tile = xr_ref[pl.ds(t * 128, 128), :]
        cdot = lax.dot_general(cen8, x_tile, (((1,), (1,)), ((), ())),
                               preferred_element_type=jnp.float32)
        d = (-2.0 * cdot + cc) + xx
        pid = t * 128 + li
        d = jnp.where(pid < N_POINTS, d, INF)
        d_ref[pl.ds(t, 1), :, :] = d[None]
        colmin = jnp.min(d, axis=1, keepdims=True)
        cm = jnp.where(it392 == t, colmin, cm)
        dmin = jnp.min(d, axis=0, keepdims=True)
        nidx = jnp.min(jnp.where(d == dmin, ci8, IBIG), axis=0,
                       keepdims=True) + CB * b
        bv = bestv_ref[pl.ds(t, 1), :]
        upd = dmin < bv
        bestv_ref[pl.ds(t, 1), :] = jnp.where(upd, dmin, bv)
        bi = besti_ref[pl.ds(t, 1), :]
        besti_ref[pl.ds(t, 1), :] = jnp.where(upd, nidx, bi)
        return cm

    cm0 = lax.fori_loop(0, NT, fill, jnp.full((CB, NT), INF))

    def ext(j, carry):
        cm, res = carry
        m8 = jnp.min(cm, axis=1, keepdims=True)
        t8 = jnp.min(jnp.where(cm == m8, it392, IBIG), axis=1,
                     keepdims=True)
        for c in range(CB):
            m_c = jnp.sum(lax.slice(m8, (c, 0), (c + 1, 1)))
            t_c = jnp.sum(lax.slice(t8, (c, 0), (c + 1, 1)))
            row = d_ref[pl.ds(t_c, 1), c, :]
            l_c = jnp.min(jnp.where(row == m_c, li, IBIG))
            g = t_c * 128 + l_c
            new_row = jnp.where(li == l_c, INF, row)
            d_ref[pl.ds(t_c, 1), c, :] = new_row
            nm = jnp.min(new_row)
            res = jnp.where((cr8 == c) & (cl8 == j), g, res)
            cm = jnp.where((is8 == c) & (it392 == t_c), nm, cm)
        return cm, res

    _, res = lax.fori_loop(0, GROUP_SIZE, ext,
                           (cm0, jnp.zeros((CB, 128), jnp.int32)))
    topk_ref[...] = res
    near_ref[...] = besti_ref[...]


def _knn(xc, xr, centers):
    return pl.pallas_call(
        _knn_body,
        grid=(NBLK,),
        in_specs=[
            pl.BlockSpec((3, NT, 128), lambda b: (0, 0, 0)),
            pl.BlockSpec((NPAD, 128), lambda b: (0, 0)),
            pl.BlockSpec((CB, 128), lambda b: (b, 0)),
        ],
        out_specs=[
            pl.BlockSpec((CB, 128), lambda b: (b, 0)),
            pl.BlockSpec((NT, 128), lambda b: (0, 0)),
        ],
        out_shape=[
            jax.ShapeDtypeStruct((NUM_GROUPS, 128), jnp.int32),
            jax.ShapeDtypeStruct((NT, 128), jnp.int32),
        ],
        scratch_shapes=[
            pltpu.VMEM((NT, CB, 128), jnp.float32),
            pltpu.VMEM((NT, 128), jnp.float32),
            pltpu.VMEM((NT, 128), jnp.int32),
        ],
    )(xc, xr, centers)


def _sc_gather(table, idx, D):
    B = idx.shape[0]
    info = plsc.get_sparse_core_info()
    NC, NS = info.num_cores, info.num_subcores
    NW = NC * NS
    bpw = B // NW
    nk = bpw // 128
    mesh = plsc.VectorSubcoreMesh(core_axis_name="c", subcore_axis_name="s")

    @functools.partial(
        pl.kernel, mesh=mesh,
        out_type=jax.ShapeDtypeStruct((B, D), jnp.float32),
        scratch_types=[
            pltpu.VMEM((bpw,), jnp.int32),
            pltpu.VMEM((2 * 128, D), jnp.float32),
            pltpu.SemaphoreType.DMA,
        ],
    )
    def g(table_hbm, idx_hbm, out_hbm, idx_v, rows_v, sem):
        wid = lax.axis_index("s") * NC + lax.axis_index("c")
        base = wid * bpw
        pltpu.sync_copy(idx_hbm.at[pl.ds(base, bpw)], idx_v)
        cps = [None] * nk
        for j in range(nk):
            cps[j] = pltpu.async_copy(
                table_hbm.at[idx_v.at[pl.ds(j * 128, 128)]],
                rows_v.at[pl.ds((j % 2) * 128, 128)], sem)
            if j >= 1:
                cps[j - 1].wait()
                pltpu.sync_copy(rows_v.at[pl.ds(((j - 1) % 2) * 128, 128)],
                                out_hbm.at[pl.ds(base + (j - 1) * 128, 128)])
        cps[nk - 1].wait()
        pltpu.sync_copy(rows_v.at[pl.ds(((nk - 1) % 2) * 128, 128)],
                        out_hbm.at[pl.ds(base + (nk - 1) * 128, 128)])

    return g(table, idx)


def _nt(a, b):
    return lax.dot_general(a, b, (((1,), (1,)), ((), ())),
                           preferred_element_type=jnp.float32)


def _nn(a, b):
    return lax.dot_general(a, b, (((1,), (0,)), ((), ())),
                           preferred_element_type=jnp.float32)


def _ln(x, g, b):
    m = jnp.mean(x, axis=-1, keepdims=True)
    v = jnp.mean((x - m) ** 2, axis=-1, keepdims=True)
    return (x - m) / jnp.sqrt(v + 1e-5) * g + b


def _bn_rows(x, g, b):
    m = jnp.mean(x, axis=0, keepdims=True)
    v = jnp.mean((x - m) ** 2, axis=0, keepdims=True)
    return (x - m) / jnp.sqrt(v + 1e-5) * g + b


def _gelu(x):
    return 0.5 * x * (1.0 + lax.erf(x * (2.0 ** -0.5)))


def _dense_body(gf_ref, pos_ref,
                c1aw_ref, c1ab_ref, bn1g_ref, bn1b_ref, c1bw_ref, c1bb_ref,
                c2aw_ref, c2ab_ref, bn2g_ref, bn2b_ref, c2bw_ref, c2bb_ref,
                ln1g_ref, ln1b_ref, qkvw_ref, qkvb_ref, fcw_ref, fcb_ref,
                ln2g_ref, ln2b_ref, m1w_ref, m1b_ref, m2w_ref, m2b_ref,
                heg_ref, heb_ref, hew1_ref, heb1_ref, hew2_ref, heb2_ref,
                hpg_ref, hpb_ref, hpw1_ref, hpb1_ref, hpw2_ref, hpb2_ref,
                out_ref):
    G, M = NUM_GROUPS, GROUP_SIZE
    NM = G * M
    gf = gf_ref[...]
    h = _nt(gf, c1aw_ref[...]) + c1ab_ref[...]
    h = jnp.maximum(_bn_rows(h, bn1g_ref[...], bn1b_ref[...]), 0.0)
    h2 = _nt(h, c1bw_ref[...]) + c1bb_ref[...]
    gmax = jnp.max(h2.reshape(G, M, 64), axis=1, keepdims=True)
    gmaxb = jnp.broadcast_to(gmax, (G, M, 64)).reshape(NM, 64)
    hc = jnp.concatenate([h2, gmaxb], axis=1)
    h3 = _nt(hc, c2aw_ref[...]) + c2ab_ref[...]
    h3 = jnp.maximum(_bn_rows(h3, bn2g_ref[...], bn2b_ref[...]), 0.0)
    h4 = _nt(h3, c2bw_ref[...]) + c2bb_ref[...]
    tokens = jnp.max(h4.reshape(G, M, 128), axis=1) + pos_ref[0]

    for i in range(2):
        hh = _ln(tokens, ln1g_ref[pl.ds(i, 1), :], ln1b_ref[pl.ds(i, 1), :])
        qkv = _nt(hh, qkvw_ref[i]) + qkvb_ref[pl.ds(i, 1), :]
        heads = []
        for hd in range(8):
            q = lax.slice(qkv, (0, hd * 16), (G, hd * 16 + 16))
            k = lax.slice(qkv, (0, 128 + hd * 16), (G, 128 + hd * 16 + 16))
            v = lax.slice(qkv, (0, 256 + hd * 16), (G, 256 + hd * 16 + 16))
            s = _nt(q, k) * (16.0 ** -0.5)
            s = s - jnp.max(s, axis=-1, keepdims=True)
            e = jnp.exp(s)
            a = e / jnp.sum(e, axis=-1, keepdims=True)
            heads.append(_nn(a, v))
        ao = jnp.concatenate(heads, axis=1)
        tokens = tokens + _nt(ao, fcw_ref[i]) + fcb_ref[pl.ds(i, 1), :]
        hh = _ln(tokens, ln2g_ref[pl.ds(i, 1), :], ln2b_ref[pl.ds(i, 1), :])
        mid = _gelu(_nt(hh, m1w_ref[i]) + m1b_ref[pl.ds(i, 1), :])
        tokens = tokens + _nt(mid, m2w_ref[i]) + m2b_ref[pl.ds(i, 1), :]

    def head(x, g_, b_, w1, b1, w2, b2):
        t = _ln(x, g_, b_)
        t = _gelu(_nt(t, w1) + b1)
        return _nt(t, w2) + b2

    e = head(tokens, heg_ref[...], heb_ref[...], hew1_ref[...],
             heb1_ref[...], hew2_ref[...], heb2_ref[...])
    p = head(tokens, hpg_ref[...], hpb_ref[...], hpw1_ref[...],
             hpb1_ref[...], hpw2_ref[...], hpb2_ref[...])
    pad = jnp.zeros((NUM_GROUPS, 96), jnp.float32)
    out_ref[...] = jnp.concatenate([e, p, pad], axis=1)


def _dense(gathered, pos_emb, *ws):
    return pl.pallas_call(
        _dense_body,
        out_shape=jax.ShapeDtypeStruct((NUM_GROUPS, 128), jnp.float32),
    )(gathered, pos_emb, *ws)


def kernel(x, features, pos_emb, ge_c1a_w, ge_c1a_b, ge_bn1_g, ge_bn1_b,
           ge_c1b_w, ge_c1b_b, ge_c2a_w, ge_c2a_b, ge_bn2_g, ge_bn2_b,
           ge_c2b_w, ge_c2b_b, blk_ln1_g, blk_ln1_b, blk_qkv_w, blk_qkv_b,
           blk_fc_w, blk_fc_b, blk_ln2_g, blk_ln2_b, blk_mlp_w1, blk_mlp_b1,
           blk_mlp_w2, blk_mlp_b2, he_ln_g, he_ln_b, he_w1, he_b1, he_w2,
           he_b2, hp_ln_g, hp_ln_b, hp_w1, hp_b1, hp_w2, hp_b2):
    xpad = jnp.pad(x, ((0, NPAD - N_POINTS), (0, 0)))
    xc = xpad.T.reshape(3, NT, 128)
    xr = jnp.pad(xpad, ((0, 0), (0, 125)))

    centers = _fps(xc)
    topk, nearest = _knn(xc, xr, centers)

    idxflat = topk[:, :GROUP_SIZE].reshape(-1)
    gathered = _sc_gather(features, idxflat, 128)

    r2 = lambda a: a.reshape(1, -1)
    ws = (ge_c1a_w, r2(ge_c1a_b), r2(ge_bn1_g), r2(ge_bn1_b),
          ge_c1b_w, r2(ge_c1b_b), ge_c2a_w, r2(ge_c2a_b),
          r2(ge_bn2_g), r2(ge_bn2_b), ge_c2b_w, r2(ge_c2b_b),
          blk_ln1_g, blk_ln1_b, blk_qkv_w, blk_qkv_b, blk_fc_w, blk_fc_b,
          blk_ln2_g, blk_ln2_b, blk_mlp_w1, blk_mlp_b1, blk_mlp_w2,
          blk_mlp_b2,
          r2(he_ln_g), r2(he_ln_b), he_w1, r2(he_b1), he_w2, r2(he_b2),
          r2(hp_ln_g), r2(hp_ln_b), hp_w1, r2(hp_b1), hp_w2, r2(hp_b2))
    table = _dense(gathered, pos_emb, *ws)

    nflat = nearest.reshape(-1)
    npad2 = jnp.pad(nflat, (0, 53248 - NPAD))
    out = _sc_gather(table, npad2, 128)
    return out[:N_POINTS, :32]


# speedup vs baseline: 7.9023x; 4.4028x over previous
import functools

import jax
import jax.numpy as jnp
from jax import lax
from jax.experimental import pallas as pl
from jax.experimental.pallas import tpu as pltpu
from jax.experimental.pallas import tpu_sc as plsc

N_POINTS = 50000
NT = 392
NPAD = NT * 128
NUM_GROUPS = 512
GROUP_SIZE = 32
CB = 8
NBLK = NUM_GROUPS // CB
INF = float("inf")
IBIG = 1 << 30


def _fps_body(xc_ref, cen_ref):
    li = lax.broadcasted_iota(jnp.int32, (1, 128), 1)
    r0 = lax.broadcasted_iota(jnp.int32, (NT, 128), 0)
    l0 = lax.broadcasted_iota(jnp.int32, (NT, 128), 1)
    fid = r0 * 128 + l0
    valid = fid < N_POINTS

    def body(i, carry):
        f, dist = carry
        r = f // 128
        l = f - r * 128
        row0 = xc_ref[0, pl.ds(r, 1), :]
        row1 = xc_ref[1, pl.ds(r, 1), :]
        row2 = xc_ref[2, pl.ds(r, 1), :]
        sel = (li == l).astype(jnp.float32)
        cx = jnp.sum(row0 * sel)
        cy = jnp.sum(row1 * sel)
        cz = jnp.sum(row2 * sel)
        lif = li.astype(jnp.float32)
        rec = (jnp.where(li == 0, cx, 0.0) + jnp.where(li == 1, cy, 0.0)
               + jnp.where(li == 2, cz, 0.0))
        cen_ref[pl.ds(i, 1), :] = rec
        dx = xc_ref[0] - cx
        dy = xc_ref[1] - cy
        dz = xc_ref[2] - cz
        d = dx * dx + dy * dy + dz * dz
        dist = jnp.where(valid, jnp.minimum(dist, d), dist)
        m = jnp.max(dist)
        f_new = jnp.min(jnp.where(dist == m, fid, IBIG))
        return f_new, dist

    dist0 = jnp.where(valid, jnp.float32(1e10), jnp.float32(-1.0))
    lax.fori_loop(0, NUM_GROUPS, body, (jnp.int32(0), dist0))


def _fps(xc):
    return pl.pallas_call(
        _fps_body,
        out_shape=jax.ShapeDtypeStruct((NUM_GROUPS, 128), jnp.float32),
    )(xc)


def _knn_body(xc_ref, xr_ref, cen_ref, topk_ref, near_ref,
              d0, d1, d2, d3, d4, d5, d6, d7, xx_ref, bestv_ref, besti_ref):
    b = pl.program_id(0)
    li = lax.broadcasted_iota(jnp.int32, (1, 128), 1)
    ci8 = lax.broadcasted_iota(jnp.int32, (CB, 1), 0)
    it392 = lax.broadcasted_iota(jnp.int32, (CB, NT), 1)
    is8 = lax.broadcasted_iota(jnp.int32, (CB, NT), 0)
    cr8 = lax.broadcasted_iota(jnp.int32, (CB, 128), 0)
    cl8 = lax.broadcasted_iota(jnp.int32, (CB, 128), 1)
    drefs = [d0, d1, d2, d3, d4, d5, d6, d7]

    cen8 = cen_ref[...]
    cx = cen_ref[:, 0:1]
    cy = cen_ref[:, 1:2]
    cz = cen_ref[:, 2:3]
    cc = (cx * cx + cy * cy) + cz * cz

    @pl.when(b == 0)
    def _init():
        x0 = xc_ref[0]
        x1 = xc_ref[1]
        x2 = xc_ref[2]
        xx = (x0 * x0 + x1 * x1) + x2 * x2
        r0 = lax.broadcasted_iota(jnp.int32, (NT, 128), 0)
        l0 = lax.broadcasted_iota(jnp.int32, (NT, 128), 1)
        fid = r0 * 128 + l0
        xx_ref[...] = jnp.where(fid < N_POINTS, xx, jnp.float32(3e38))
        bestv_ref[...] = jnp.full((NT, 128), INF)
        besti_ref[...] = jnp.zeros((NT, 128), jnp.int32)

    def fill(s, cm):
        xs = xr_ref[pl.ds(s * 1024, 1024), :]
        cdot = lax.dot_general(cen8, xs, (((1,), (1,)), ((), ())),
                               preferred_element_type=jnp.float32)
        for k in range(8):
            t = s * 8 + k
            dk = lax.slice(cdot, (0, k * 128), (CB, (k + 1) * 128))
            xxr = xx_ref[pl.ds(t, 1), :]
            d = (-2.0 * dk + cc) + xxr
            for c in range(CB):
                drefs[c][pl.ds(t, 1), :] = lax.slice(d, (c, 0), (c + 1, 128))
            colmin = jnp.min(d, axis=1, keepdims=True)
            cm = jnp.where(it392 == t, colmin, cm)
            dmin = jnp.min(d, axis=0, keepdims=True)
            nidx = jnp.min(jnp.where(d == dmin, ci8, IBIG), axis=0,
                           keepdims=True) + CB * b
            bv = bestv_ref[pl.ds(t, 1), :]
            upd = dmin < bv
            bestv_ref[pl.ds(t, 1), :] = jnp.where(upd, dmin, bv)
            bi = besti_ref[pl.ds(t, 1), :]
            besti_ref[pl.ds(t, 1), :] = jnp.where(upd, nidx, bi)
        return cm

    cm0 = lax.fori_loop(0, NT // 8, fill, jnp.full((CB, NT), INF))

    def ext(j, carry):
        cm, res = carry
        m8 = jnp.min(cm, axis=1, keepdims=True)
        t8 = jnp.min(jnp.where(cm == m8, it392, IBIG), axis=1,
                     keepdims=True)
        for c in range(CB):
            t_c = jnp.sum(lax.slice(t8, (c, 0), (c + 1, 1)))
            row = drefs[c][pl.ds(t_c, 1), :]
            rmv = jnp.min(row, axis=1, keepdims=True)
            lv = jnp.min(jnp.where(row == rmv, li, IBIG), axis=1,
                         keepdims=True)
            new_row = jnp.where(li == lv, INF, row)
            drefs[c][pl.ds(t_c, 1), :] = new_row
            nmv = jnp.min(new_row, axis=1, keepdims=True)
            g = t_c * 128 + lv
            res = jnp.where((cr8 == c) & (cl8 == j), g, res)
            cm = jnp.where((is8 == c) & (it392 == t_c), nmv, cm)
        return cm, res

    _, res = lax.fori_loop(0, GROUP_SIZE, ext,
                           (cm0, jnp.zeros((CB, 128), jnp.int32)))
    topk_ref[...] = res
    near_ref[...] = besti_ref[...]


def _knn(xc, xr, centers):
    return pl.pallas_call(
        _knn_body,
        grid=(NBLK,),
        in_specs=[
            pl.BlockSpec((3, NT, 128), lambda b: (0, 0, 0)),
            pl.BlockSpec((NPAD, 128), lambda b: (0, 0)),
            pl.BlockSpec((CB, 128), lambda b: (b, 0)),
        ],
        out_specs=[
            pl.BlockSpec((CB, 128), lambda b: (b, 0)),
            pl.BlockSpec((NT, 128), lambda b: (0, 0)),
        ],
        out_shape=[
            jax.ShapeDtypeStruct((NUM_GROUPS, 128), jnp.int32),
            jax.ShapeDtypeStruct((NT, 128), jnp.int32),
        ],
        scratch_shapes=(
            [pltpu.VMEM((NT, 128), jnp.float32) for _ in range(CB)]
            + [pltpu.VMEM((NT, 128), jnp.float32),
               pltpu.VMEM((NT, 128), jnp.float32),
               pltpu.VMEM((NT, 128), jnp.int32)]
        ),
    )(xc, xr, centers)


def _sc_gather(table, idx, D):
    B = idx.shape[0]
    info = plsc.get_sparse_core_info()
    NC, NS = info.num_cores, info.num_subcores
    NW = NC * NS
    bpw = B // NW
    nk = bpw // 128
    mesh = plsc.VectorSubcoreMesh(core_axis_name="c", subcore_axis_name="s")

    @functools.partial(
        pl.kernel, mesh=mesh,
        out_type=jax.ShapeDtypeStruct((B, D), jnp.float32),
        scratch_types=[
            pltpu.VMEM((bpw,), jnp.int32),
            pltpu.VMEM((2 * 128, D), jnp.float32),
            pltpu.SemaphoreType.DMA,
        ],
    )
    def g(table_hbm, idx_hbm, out_hbm, idx_v, rows_v, sem):
        wid = lax.axis_index("s") * NC + lax.axis_index("c")
        base = wid * bpw
        pltpu.sync_copy(idx_hbm.at[pl.ds(base, bpw)], idx_v)
        cps = [None] * nk
        for j in range(nk):
            cps[j] = pltpu.async_copy(
                table_hbm.at[idx_v.at[pl.ds(j * 128, 128)]],
                rows_v.at[pl.ds((j % 2) * 128, 128)], sem)
            if j >= 1:
                cps[j - 1].wait()
                pltpu.sync_copy(rows_v.at[pl.ds(((j - 1) % 2) * 128, 128)],
                                out_hbm.at[pl.ds(base + (j - 1) * 128, 128)])
        cps[nk - 1].wait()
        pltpu.sync_copy(rows_v.at[pl.ds(((nk - 1) % 2) * 128, 128)],
                        out_hbm.at[pl.ds(base + (nk - 1) * 128, 128)])

    return g(table, idx)


def _nt(a, b):
    return lax.dot_general(a, b, (((1,), (1,)), ((), ())),
                           preferred_element_type=jnp.float32)


def _nn(a, b):
    return lax.dot_general(a, b, (((1,), (0,)), ((), ())),
                           preferred_element_type=jnp.float32)


def _ln(x, g, b):
    m = jnp.mean(x, axis=-1, keepdims=True)
    v = jnp.mean((x - m) ** 2, axis=-1, keepdims=True)
    return (x - m) / jnp.sqrt(v + 1e-5) * g + b


def _bn_rows(x, g, b):
    m = jnp.mean(x, axis=0, keepdims=True)
    v = jnp.mean((x - m) ** 2, axis=0, keepdims=True)
    return (x - m) / jnp.sqrt(v + 1e-5) * g + b


def _gelu(x):
    return 0.5 * x * (1.0 + lax.erf(x * (2.0 ** -0.5)))


def _dense_body(gf_ref, pos_ref,
                c1aw_ref, c1ab_ref, bn1g_ref, bn1b_ref, c1bw_ref, c1bb_ref,
                c2aw_ref, c2ab_ref, bn2g_ref, bn2b_ref, c2bw_ref, c2bb_ref,
                ln1g_ref, ln1b_ref, qkvw_ref, qkvb_ref, fcw_ref, fcb_ref,
                ln2g_ref, ln2b_ref, m1w_ref, m1b_ref, m2w_ref, m2b_ref,
                heg_ref, heb_ref, hew1_ref, heb1_ref, hew2_ref, heb2_ref,
                hpg_ref, hpb_ref, hpw1_ref, hpb1_ref, hpw2_ref, hpb2_ref,
                out_ref):
    G, M = NUM_GROUPS, GROUP_SIZE
    NM = G * M
    gf = gf_ref[...]
    h = _nt(gf, c1aw_ref[...]) + c1ab_ref[...]
    h = jnp.maximum(_bn_rows(h, bn1g_ref[...], bn1b_ref[...]), 0.0)
    h2 = _nt(h, c1bw_ref[...]) + c1bb_ref[...]
    gmax = jnp.max(h2.reshape(G, M, 64), axis=1, keepdims=True)
    gmaxb = jnp.broadcast_to(gmax, (G, M, 64)).reshape(NM, 64)
    hc = jnp.concatenate([h2, gmaxb], axis=1)
    h3 = _nt(hc, c2aw_ref[...]) + c2ab_ref[...]
    h3 = jnp.maximum(_bn_rows(h3, bn2g_ref[...], bn2b_ref[...]), 0.0)
    h4 = _nt(h3, c2bw_ref[...]) + c2bb_ref[...]
    tokens = jnp.max(h4.reshape(G, M, 128), axis=1) + pos_ref[0]

    for i in range(2):
        hh = _ln(tokens, ln1g_ref[pl.ds(i, 1), :], ln1b_ref[pl.ds(i, 1), :])
        qkv = _nt(hh, qkvw_ref[i]) + qkvb_ref[pl.ds(i, 1), :]
        heads = []
        for hd in range(8):
            q = lax.slice(qkv, (0, hd * 16), (G, hd * 16 + 16))
            k = lax.slice(qkv, (0, 128 + hd * 16), (G, 128 + hd * 16 + 16))
            v = lax.slice(qkv, (0, 256 + hd * 16), (G, 256 + hd * 16 + 16))
            s = _nt(q, k) * (16.0 ** -0.5)
            s = s - jnp.max(s, axis=-1, keepdims=True)
            e = jnp.exp(s)
            a = e / jnp.sum(e, axis=-1, keepdims=True)
            heads.append(_nn(a, v))
        ao = jnp.concatenate(heads, axis=1)
        tokens = tokens + _nt(ao, fcw_ref[i]) + fcb_ref[pl.ds(i, 1), :]
        hh = _ln(tokens, ln2g_ref[pl.ds(i, 1), :], ln2b_ref[pl.ds(i, 1), :])
        mid = _gelu(_nt(hh, m1w_ref[i]) + m1b_ref[pl.ds(i, 1), :])
        tokens = tokens + _nt(mid, m2w_ref[i]) + m2b_ref[pl.ds(i, 1), :]

    def head(x, g_, b_, w1, b1, w2, b2):
        t = _ln(x, g_, b_)
        t = _gelu(_nt(t, w1) + b1)
        return _nt(t, w2) + b2

    e = head(tokens, heg_ref[...], heb_ref[...], hew1_ref[...],
             heb1_ref[...], hew2_ref[...], heb2_ref[...])
    p = head(tokens, hpg_ref[...], hpb_ref[...], hpw1_ref[...],
             hpb1_ref[...], hpw2_ref[...], hpb2_ref[...])
    pad = jnp.zeros((NUM_GROUPS, 96), jnp.float32)
    out_ref[...] = jnp.concatenate([e, p, pad], axis=1)


def _dense(gathered, pos_emb, *ws):
    return pl.pallas_call(
        _dense_body,
        out_shape=jax.ShapeDtypeStruct((NUM_GROUPS, 128), jnp.float32),
    )(gathered, pos_emb, *ws)


def kernel(x, features, pos_emb, ge_c1a_w, ge_c1a_b, ge_bn1_g, ge_bn1_b,
           ge_c1b_w, ge_c1b_b, ge_c2a_w, ge_c2a_b, ge_bn2_g, ge_bn2_b,
           ge_c2b_w, ge_c2b_b, blk_ln1_g, blk_ln1_b, blk_qkv_w, blk_qkv_b,
           blk_fc_w, blk_fc_b, blk_ln2_g, blk_ln2_b, blk_mlp_w1, blk_mlp_b1,
           blk_mlp_w2, blk_mlp_b2, he_ln_g, he_ln_b, he_w1, he_b1, he_w2,
           he_b2, hp_ln_g, hp_ln_b, hp_w1, hp_b1, hp_w2, hp_b2):
    xpad = jnp.pad(x, ((0, NPAD - N_POINTS), (0, 0)))
    xc = xpad.T.reshape(3, NT, 128)
    xr = jnp.pad(xpad, ((0, 0), (0, 125)))

    centers = _fps(xc)
    topk, nearest = _knn(xc, xr, centers)

    idxflat = topk[:, :GROUP_SIZE].reshape(-1)
    gathered = _sc_gather(features, idxflat, 128)

    r2 = lambda a: a.reshape(1, -1)
    ws = (ge_c1a_w, r2(ge_c1a_b), r2(ge_bn1_g), r2(ge_bn1_b),
          ge_c1b_w, r2(ge_c1b_b), ge_c2a_w, r2(ge_c2a_b),
          r2(ge_bn2_g), r2(ge_bn2_b), ge_c2b_w, r2(ge_c2b_b),
          blk_ln1_g, blk_ln1_b, blk_qkv_w, blk_qkv_b, blk_fc_w, blk_fc_b,
          blk_ln2_g, blk_ln2_b, blk_mlp_w1, blk_mlp_b1, blk_mlp_w2,
          blk_mlp_b2,
          r2(he_ln_g), r2(he_ln_b), he_w1, r2(he_b1), he_w2, r2(he_b2),
          r2(hp_ln_g), r2(hp_ln_b), hp_w1, r2(hp_b1), hp_w2, r2(hp_b2))
    table = _dense(gathered, pos_emb, *ws)

    nflat = nearest.reshape(-1)
    npad2 = jnp.pad(nflat, (0, 53248 - NPAD))
    out = _sc_gather(table, npad2, 128)
    return out[:N_POINTS, :32]


# speedup vs baseline: 12.4022x; 1.5694x over previous
import functools

import jax
import jax.numpy as jnp
from jax import lax
from jax.experimental import pallas as pl
from jax.experimental.pallas import tpu as pltpu
from jax.experimental.pallas import tpu_sc as plsc

N_POINTS = 50000
NT = 392
NPAD = NT * 128
NUM_GROUPS = 512
GROUP_SIZE = 32
CB = 16
NBLK = NUM_GROUPS // CB
INF = float("inf")
IBIG = 1 << 30


def _fps_body(xc_ref, cen_ref):
    li = lax.broadcasted_iota(jnp.int32, (1, 128), 1)
    r0 = lax.broadcasted_iota(jnp.int32, (NT, 128), 0)
    l0 = lax.broadcasted_iota(jnp.int32, (NT, 128), 1)
    fid = r0 * 128 + l0
    valid = fid < N_POINTS

    def body(i, carry):
        f, dist = carry
        r = f // 128
        l = f - r * 128
        row0 = xc_ref[0, pl.ds(r, 1), :]
        row1 = xc_ref[1, pl.ds(r, 1), :]
        row2 = xc_ref[2, pl.ds(r, 1), :]
        sel = (li == l).astype(jnp.float32)
        cx = jnp.sum(row0 * sel)
        cy = jnp.sum(row1 * sel)
        cz = jnp.sum(row2 * sel)
        lif = li.astype(jnp.float32)
        rec = (jnp.where(li == 0, cx, 0.0) + jnp.where(li == 1, cy, 0.0)
               + jnp.where(li == 2, cz, 0.0))
        cen_ref[pl.ds(i, 1), :] = rec
        dx = xc_ref[0] - cx
        dy = xc_ref[1] - cy
        dz = xc_ref[2] - cz
        d = dx * dx + dy * dy + dz * dz
        dist = jnp.where(valid, jnp.minimum(dist, d), dist)
        m = jnp.max(dist)
        f_new = jnp.min(jnp.where(dist == m, fid, IBIG))
        return f_new, dist

    dist0 = jnp.where(valid, jnp.float32(1e10), jnp.float32(-1.0))
    lax.fori_loop(0, NUM_GROUPS, body, (jnp.int32(0), dist0))


def _fps(xc):
    return pl.pallas_call(
        _fps_body,
        out_shape=jax.ShapeDtypeStruct((NUM_GROUPS, 128), jnp.float32),
    )(xc)


def _knn_body(xc_ref, xr_ref, cen_ref, topk_ref, near_ref,
              *refs):
    b = pl.program_id(0)
    li = lax.broadcasted_iota(jnp.int32, (1, 128), 1)
    ci8 = lax.broadcasted_iota(jnp.int32, (CB, 1), 0)
    it392 = lax.broadcasted_iota(jnp.int32, (CB, NT), 1)
    is8 = lax.broadcasted_iota(jnp.int32, (CB, NT), 0)
    cr8 = lax.broadcasted_iota(jnp.int32, (CB, 128), 0)
    cl8 = lax.broadcasted_iota(jnp.int32, (CB, 128), 1)
    drefs = list(refs[:CB])
    xx_ref, bestv_ref, besti_ref = refs[CB], refs[CB + 1], refs[CB + 2]

    cen8 = cen_ref[...]
    cx = cen_ref[:, 0:1]
    cy = cen_ref[:, 1:2]
    cz = cen_ref[:, 2:3]
    cc = (cx * cx + cy * cy) + cz * cz

    @pl.when(b == 0)
    def _init():
        x0 = xc_ref[0]
        x1 = xc_ref[1]
        x2 = xc_ref[2]
        xx = (x0 * x0 + x1 * x1) + x2 * x2
        r0 = lax.broadcasted_iota(jnp.int32, (NT, 128), 0)
        l0 = lax.broadcasted_iota(jnp.int32, (NT, 128), 1)
        fid = r0 * 128 + l0
        xx_ref[...] = jnp.where(fid < N_POINTS, xx, jnp.float32(3e38))
        bestv_ref[...] = jnp.full((NT, 128), INF)
        besti_ref[...] = jnp.zeros((NT, 128), jnp.int32)

    def fill(s, cm):
        xs = xr_ref[pl.ds(s * 1024, 1024), :]
        cdot = lax.dot_general(cen8, xs, (((1,), (1,)), ((), ())),
                               preferred_element_type=jnp.float32)
        for k in range(8):
            t = s * 8 + k
            dk = lax.slice(cdot, (0, k * 128), (CB, (k + 1) * 128))
            xxr = xx_ref[pl.ds(t, 1), :]
            d = (-2.0 * dk + cc) + xxr
            for c in range(CB):
                drefs[c][pl.ds(t, 1), :] = lax.slice(d, (c, 0), (c + 1, 128))
            colmin = jnp.min(d, axis=1, keepdims=True)
            cm = jnp.where(it392 == t, colmin, cm)
            dmin = jnp.min(d, axis=0, keepdims=True)
            nidx = jnp.min(jnp.where(d == dmin, ci8, IBIG), axis=0,
                           keepdims=True) + CB * b
            bv = bestv_ref[pl.ds(t, 1), :]
            upd = dmin < bv
            bestv_ref[pl.ds(t, 1), :] = jnp.where(upd, dmin, bv)
            bi = besti_ref[pl.ds(t, 1), :]
            besti_ref[pl.ds(t, 1), :] = jnp.where(upd, nidx, bi)
        return cm

    cm0 = lax.fori_loop(0, NT // 8, fill, jnp.full((CB, NT), INF))

    def ext(j, carry):
        cm, res = carry
        m8 = jnp.min(cm, axis=1, keepdims=True)
        t8 = jnp.min(jnp.where(cm == m8, it392, IBIG), axis=1,
                     keepdims=True)
        for c in range(CB):
            t_c = jnp.sum(lax.slice(t8, (c, 0), (c + 1, 1)))
            row = drefs[c][pl.ds(t_c, 1), :]
            rmv = jnp.min(row, axis=1, keepdims=True)
            lv = jnp.min(jnp.where(row == rmv, li, IBIG), axis=1,
                         keepdims=True)
            new_row = jnp.where(li == lv, INF, row)
            drefs[c][pl.ds(t_c, 1), :] = new_row
            nmv = jnp.min(new_row, axis=1, keepdims=True)
            g = t_c * 128 + lv
            res = jnp.where((cr8 == c) & (cl8 == j), g, res)
            cm = jnp.where((is8 == c) & (it392 == t_c), nmv, cm)
        return cm, res

    _, res = lax.fori_loop(0, GROUP_SIZE, ext,
                           (cm0, jnp.zeros((CB, 128), jnp.int32)))
    topk_ref[...] = res
    near_ref[...] = besti_ref[...]


def _knn(xc, xr, centers):
    return pl.pallas_call(
        _knn_body,
        grid=(NBLK,),
        in_specs=[
            pl.BlockSpec((3, NT, 128), lambda b: (0, 0, 0)),
            pl.BlockSpec((NPAD, 128), lambda b: (0, 0)),
            pl.BlockSpec((CB, 128), lambda b: (b, 0)),
        ],
        out_specs=[
            pl.BlockSpec((CB, 128), lambda b: (b, 0)),
            pl.BlockSpec((NT, 128), lambda b: (0, 0)),
        ],
        out_shape=[
            jax.ShapeDtypeStruct((NUM_GROUPS, 128), jnp.int32),
            jax.ShapeDtypeStruct((NT, 128), jnp.int32),
        ],
        scratch_shapes=(
            [pltpu.VMEM((NT, 128), jnp.float32) for _ in range(CB)]
            + [pltpu.VMEM((NT, 128), jnp.float32),
               pltpu.VMEM((NT, 128), jnp.float32),
               pltpu.VMEM((NT, 128), jnp.int32)]
        ),
    )(xc, xr, centers)


def _sc_gather(table, idx, D):
    B = idx.shape[0]
    info = plsc.get_sparse_core_info()
    NC, NS = info.num_cores, info.num_subcores
    NW = NC * NS
    bpw = B // NW
    nk = bpw // 128
    mesh = plsc.VectorSubcoreMesh(core_axis_name="c", subcore_axis_name="s")

    @functools.partial(
        pl.kernel, mesh=mesh,
        out_type=jax.ShapeDtypeStruct((B, D), jnp.float32),
        scratch_types=[
            pltpu.VMEM((bpw,), jnp.int32),
            pltpu.VMEM((2 * 128, D), jnp.float32),
            pltpu.SemaphoreType.DMA,
        ],
    )
    def g(table_hbm, idx_hbm, out_hbm, idx_v, rows_v, sem):
        wid = lax.axis_index("s") * NC + lax.axis_index("c")
        base = wid * bpw
        pltpu.sync_copy(idx_hbm.at[pl.ds(base, bpw)], idx_v)
        cps = [None] * nk
        for j in range(nk):
            cps[j] = pltpu.async_copy(
                table_hbm.at[idx_v.at[pl.ds(j * 128, 128)]],
                rows_v.at[pl.ds((j % 2) * 128, 128)], sem)
            if j >= 1:
                cps[j - 1].wait()
                pltpu.sync_copy(rows_v.at[pl.ds(((j - 1) % 2) * 128, 128)],
                                out_hbm.at[pl.ds(base + (j - 1) * 128, 128)])
        cps[nk - 1].wait()
        pltpu.sync_copy(rows_v.at[pl.ds(((nk - 1) % 2) * 128, 128)],
                        out_hbm.at[pl.ds(base + (nk - 1) * 128, 128)])

    return g(table, idx)


def _nt(a, b):
    return lax.dot_general(a, b, (((1,), (1,)), ((), ())),
                           preferred_element_type=jnp.float32)


def _nn(a, b):
    return lax.dot_general(a, b, (((1,), (0,)), ((), ())),
                           preferred_element_type=jnp.float32)


def _ln(x, g, b):
    m = jnp.mean(x, axis=-1, keepdims=True)
    v = jnp.mean((x - m) ** 2, axis=-1, keepdims=True)
    return (x - m) / jnp.sqrt(v + 1e-5) * g + b


def _bn_rows(x, g, b):
    m = jnp.mean(x, axis=0, keepdims=True)
    v = jnp.mean((x - m) ** 2, axis=0, keepdims=True)
    return (x - m) / jnp.sqrt(v + 1e-5) * g + b


def _gelu(x):
    return 0.5 * x * (1.0 + lax.erf(x * (2.0 ** -0.5)))


def _dense_body(gf_ref, pos_ref,
                c1aw_ref, c1ab_ref, bn1g_ref, bn1b_ref, c1bw_ref, c1bb_ref,
                c2aw_ref, c2ab_ref, bn2g_ref, bn2b_ref, c2bw_ref, c2bb_ref,
                ln1g_ref, ln1b_ref, qkvw_ref, qkvb_ref, fcw_ref, fcb_ref,
                ln2g_ref, ln2b_ref, m1w_ref, m1b_ref, m2w_ref, m2b_ref,
                heg_ref, heb_ref, hew1_ref, heb1_ref, hew2_ref, heb2_ref,
                hpg_ref, hpb_ref, hpw1_ref, hpb1_ref, hpw2_ref, hpb2_ref,
                out_ref):
    G, M = NUM_GROUPS, GROUP_SIZE
    NM = G * M
    gf = gf_ref[...]
    h = _nt(gf, c1aw_ref[...]) + c1ab_ref[...]
    h = jnp.maximum(_bn_rows(h, bn1g_ref[...], bn1b_ref[...]), 0.0)
    h2 = _nt(h, c1bw_ref[...]) + c1bb_ref[...]
    gmax = jnp.max(h2.reshape(G, M, 64), axis=1, keepdims=True)
    gmaxb = jnp.broadcast_to(gmax, (G, M, 64)).reshape(NM, 64)
    hc = jnp.concatenate([h2, gmaxb], axis=1)
    h3 = _nt(hc, c2aw_ref[...]) + c2ab_ref[...]
    h3 = jnp.maximum(_bn_rows(h3, bn2g_ref[...], bn2b_ref[...]), 0.0)
    h4 = _nt(h3, c2bw_ref[...]) + c2bb_ref[...]
    tokens = jnp.max(h4.reshape(G, M, 128), axis=1) + pos_ref[0]

    for i in range(2):
        hh = _ln(tokens, ln1g_ref[pl.ds(i, 1), :], ln1b_ref[pl.ds(i, 1), :])
        qkv = _nt(hh, qkvw_ref[i]) + qkvb_ref[pl.ds(i, 1), :]
        heads = []
        for hd in range(8):
            q = lax.slice(qkv, (0, hd * 16), (G, hd * 16 + 16))
            k = lax.slice(qkv, (0, 128 + hd * 16), (G, 128 + hd * 16 + 16))
            v = lax.slice(qkv, (0, 256 + hd * 16), (G, 256 + hd * 16 + 16))
            s = _nt(q, k) * (16.0 ** -0.5)
            s = s - jnp.max(s, axis=-1, keepdims=True)
            e = jnp.exp(s)
            a = e / jnp.sum(e, axis=-1, keepdims=True)
            heads.append(_nn(a, v))
        ao = jnp.concatenate(heads, axis=1)
        tokens = tokens + _nt(ao, fcw_ref[i]) + fcb_ref[pl.ds(i, 1), :]
        hh = _ln(tokens, ln2g_ref[pl.ds(i, 1), :], ln2b_ref[pl.ds(i, 1), :])
        mid = _gelu(_nt(hh, m1w_ref[i]) + m1b_ref[pl.ds(i, 1), :])
        tokens = tokens + _nt(mid, m2w_ref[i]) + m2b_ref[pl.ds(i, 1), :]

    def head(x, g_, b_, w1, b1, w2, b2):
        t = _ln(x, g_, b_)
        t = _gelu(_nt(t, w1) + b1)
        return _nt(t, w2) + b2

    e = head(tokens, heg_ref[...], heb_ref[...], hew1_ref[...],
             heb1_ref[...], hew2_ref[...], heb2_ref[...])
    p = head(tokens, hpg_ref[...], hpb_ref[...], hpw1_ref[...],
             hpb1_ref[...], hpw2_ref[...], hpb2_ref[...])
    pad = jnp.zeros((NUM_GROUPS, 96), jnp.float32)
    out_ref[...] = jnp.concatenate([e, p, pad], axis=1)


def _dense(gathered, pos_emb, *ws):
    return pl.pallas_call(
        _dense_body,
        out_shape=jax.ShapeDtypeStruct((NUM_GROUPS, 128), jnp.float32),
    )(gathered, pos_emb, *ws)


def kernel(x, features, pos_emb, ge_c1a_w, ge_c1a_b, ge_bn1_g, ge_bn1_b,
           ge_c1b_w, ge_c1b_b, ge_c2a_w, ge_c2a_b, ge_bn2_g, ge_bn2_b,
           ge_c2b_w, ge_c2b_b, blk_ln1_g, blk_ln1_b, blk_qkv_w, blk_qkv_b,
           blk_fc_w, blk_fc_b, blk_ln2_g, blk_ln2_b, blk_mlp_w1, blk_mlp_b1,
           blk_mlp_w2, blk_mlp_b2, he_ln_g, he_ln_b, he_w1, he_b1, he_w2,
           he_b2, hp_ln_g, hp_ln_b, hp_w1, hp_b1, hp_w2, hp_b2):
    xpad = jnp.pad(x, ((0, NPAD - N_POINTS), (0, 0)))
    xc = xpad.T.reshape(3, NT, 128)
    xr = jnp.pad(xpad, ((0, 0), (0, 125)))

    centers = _fps(xc)
    topk, nearest = _knn(xc, xr, centers)

    idxflat = topk[:, :GROUP_SIZE].reshape(-1)
    gathered = _sc_gather(features, idxflat, 128)

    r2 = lambda a: a.reshape(1, -1)
    ws = (ge_c1a_w, r2(ge_c1a_b), r2(ge_bn1_g), r2(ge_bn1_b),
          ge_c1b_w, r2(ge_c1b_b), ge_c2a_w, r2(ge_c2a_b),
          r2(ge_bn2_g), r2(ge_bn2_b), ge_c2b_w, r2(ge_c2b_b),
          blk_ln1_g, blk_ln1_b, blk_qkv_w, blk_qkv_b, blk_fc_w, blk_fc_b,
          blk_ln2_g, blk_ln2_b, blk_mlp_w1, blk_mlp_b1, blk_mlp_w2,
          blk_mlp_b2,
          r2(he_ln_g), r2(he_ln_b), he_w1, r2(he_b1), he_w2, r2(he_b2),
          r2(hp_ln_g), r2(hp_ln_b), hp_w1, r2(hp_b1), hp_w2, r2(hp_b2))
    table = _dense(gathered, pos_emb, *ws)

    nflat = nearest.reshape(-1)
    npad2 = jnp.pad(nflat, (0, 53248 - NPAD))
    out = _sc_gather(table, npad2, 128)
    return out[:N_POINTS, :32]


# speedup vs baseline: 17.1062x; 1.3793x over previous
import functools

import jax
import jax.numpy as jnp
from jax import lax
from jax.experimental import pallas as pl
from jax.experimental.pallas import tpu as pltpu
from jax.experimental.pallas import tpu_sc as plsc

N_POINTS = 50000
NT = 392
NPAD = NT * 128
NUM_GROUPS = 512
GROUP_SIZE = 32
CB = 32
NBLK = NUM_GROUPS // CB
INF = float("inf")
IBIG = 1 << 30


def _fps_body(xc_ref, cen_ref):
    li = lax.broadcasted_iota(jnp.int32, (1, 128), 1)
    r0 = lax.broadcasted_iota(jnp.int32, (NT, 128), 0)
    l0 = lax.broadcasted_iota(jnp.int32, (NT, 128), 1)
    fid = r0 * 128 + l0
    valid = fid < N_POINTS

    def body(i, carry):
        f, dist = carry
        r = f // 128
        l = f - r * 128
        row0 = xc_ref[0, pl.ds(r, 1), :]
        row1 = xc_ref[1, pl.ds(r, 1), :]
        row2 = xc_ref[2, pl.ds(r, 1), :]
        sel = (li == l).astype(jnp.float32)
        cx = jnp.sum(row0 * sel)
        cy = jnp.sum(row1 * sel)
        cz = jnp.sum(row2 * sel)
        lif = li.astype(jnp.float32)
        rec = (jnp.where(li == 0, cx, 0.0) + jnp.where(li == 1, cy, 0.0)
               + jnp.where(li == 2, cz, 0.0))
        cen_ref[pl.ds(i, 1), :] = rec
        dx = xc_ref[0] - cx
        dy = xc_ref[1] - cy
        dz = xc_ref[2] - cz
        d = dx * dx + dy * dy + dz * dz
        dist = jnp.where(valid, jnp.minimum(dist, d), dist)
        m = jnp.max(dist)
        f_new = jnp.min(jnp.where(dist == m, fid, IBIG))
        return f_new, dist

    dist0 = jnp.where(valid, jnp.float32(1e10), jnp.float32(-1.0))
    lax.fori_loop(0, NUM_GROUPS, body, (jnp.int32(0), dist0))


def _fps(xc):
    return pl.pallas_call(
        _fps_body,
        out_shape=jax.ShapeDtypeStruct((NUM_GROUPS, 128), jnp.float32),
    )(xc)


def _knn_body(xc_ref, xr_ref, cen_ref, topk_ref, near_ref,
              *refs):
    b = pl.program_id(0)
    li = lax.broadcasted_iota(jnp.int32, (1, 128), 1)
    ci8 = lax.broadcasted_iota(jnp.int32, (CB, 1), 0)
    it392 = lax.broadcasted_iota(jnp.int32, (CB, NT), 1)
    is8 = lax.broadcasted_iota(jnp.int32, (CB, NT), 0)
    cr8 = lax.broadcasted_iota(jnp.int32, (CB, 128), 0)
    cl8 = lax.broadcasted_iota(jnp.int32, (CB, 128), 1)
    drefs = list(refs[:CB])
    xx_ref, bestv_ref, besti_ref = refs[CB], refs[CB + 1], refs[CB + 2]

    cen8 = cen_ref[...]
    cx = cen_ref[:, 0:1]
    cy = cen_ref[:, 1:2]
    cz = cen_ref[:, 2:3]
    cc = (cx * cx + cy * cy) + cz * cz

    @pl.when(b == 0)
    def _init():
        x0 = xc_ref[0]
        x1 = xc_ref[1]
        x2 = xc_ref[2]
        xx = (x0 * x0 + x1 * x1) + x2 * x2
        r0 = lax.broadcasted_iota(jnp.int32, (NT, 128), 0)
        l0 = lax.broadcasted_iota(jnp.int32, (NT, 128), 1)
        fid = r0 * 128 + l0
        xx_ref[...] = jnp.where(fid < N_POINTS, xx, jnp.float32(3e38))
        bestv_ref[...] = jnp.full((NT, 128), INF)
        besti_ref[...] = jnp.zeros((NT, 128), jnp.int32)

    def fill(s, cm):
        xs = xr_ref[pl.ds(s * 1024, 1024), :]
        cdot = lax.dot_general(cen8, xs, (((1,), (1,)), ((), ())),
                               preferred_element_type=jnp.float32)
        for k in range(8):
            t = s * 8 + k
            dk = lax.slice(cdot, (0, k * 128), (CB, (k + 1) * 128))
            xxr = xx_ref[pl.ds(t, 1), :]
            d = (-2.0 * dk + cc) + xxr
            for c in range(CB):
                drefs[c][pl.ds(t, 1), :] = lax.slice(d, (c, 0), (c + 1, 128))
            colmin = jnp.min(d, axis=1, keepdims=True)
            cm = jnp.where(it392 == t, colmin, cm)
            dmin = jnp.min(d, axis=0, keepdims=True)
            nidx = jnp.min(jnp.where(d == dmin, ci8, IBIG), axis=0,
                           keepdims=True) + CB * b
            bv = bestv_ref[pl.ds(t, 1), :]
            upd = dmin < bv
            bestv_ref[pl.ds(t, 1), :] = jnp.where(upd, dmin, bv)
            bi = besti_ref[pl.ds(t, 1), :]
            besti_ref[pl.ds(t, 1), :] = jnp.where(upd, nidx, bi)
        return cm

    cm0 = lax.fori_loop(0, NT // 8, fill, jnp.full((CB, NT), INF))

    def ext(j, carry):
        cm, res = carry
        m8 = jnp.min(cm, axis=1, keepdims=True)
        t8 = jnp.min(jnp.where(cm == m8, it392, IBIG), axis=1,
                     keepdims=True)
        for c in range(CB):
            t_c = jnp.sum(lax.slice(t8, (c, 0), (c + 1, 1)))
            row = drefs[c][pl.ds(t_c, 1), :]
            rmv = jnp.min(row, axis=1, keepdims=True)
            lv = jnp.min(jnp.where(row == rmv, li, IBIG), axis=1,
                         keepdims=True)
            new_row = jnp.where(li == lv, INF, row)
            drefs[c][pl.ds(t_c, 1), :] = new_row
            nmv = jnp.min(new_row, axis=1, keepdims=True)
            g = t_c * 128 + lv
            res = jnp.where((cr8 == c) & (cl8 == j), g, res)
            cm = jnp.where((is8 == c) & (it392 == t_c), nmv, cm)
        return cm, res

    _, res = lax.fori_loop(0, GROUP_SIZE, ext,
                           (cm0, jnp.zeros((CB, 128), jnp.int32)))
    topk_ref[...] = res
    near_ref[...] = besti_ref[...]


def _knn(xc, xr, centers):
    return pl.pallas_call(
        _knn_body,
        grid=(NBLK,),
        in_specs=[
            pl.BlockSpec((3, NT, 128), lambda b: (0, 0, 0)),
            pl.BlockSpec((NPAD, 128), lambda b: (0, 0)),
            pl.BlockSpec((CB, 128), lambda b: (b, 0)),
        ],
        out_specs=[
            pl.BlockSpec((CB, 128), lambda b: (b, 0)),
            pl.BlockSpec((NT, 128), lambda b: (0, 0)),
        ],
        out_shape=[
            jax.ShapeDtypeStruct((NUM_GROUPS, 128), jnp.int32),
            jax.ShapeDtypeStruct((NT, 128), jnp.int32),
        ],
        scratch_shapes=(
            [pltpu.VMEM((NT, 128), jnp.float32) for _ in range(CB)]
            + [pltpu.VMEM((NT, 128), jnp.float32),
               pltpu.VMEM((NT, 128), jnp.float32),
               pltpu.VMEM((NT, 128), jnp.int32)]
        ),
    )(xc, xr, centers)


def _sc_gather(table, idx, D):
    B = idx.shape[0]
    info = plsc.get_sparse_core_info()
    NC, NS = info.num_cores, info.num_subcores
    NW = NC * NS
    bpw = B // NW
    nk = bpw // 128
    mesh = plsc.VectorSubcoreMesh(core_axis_name="c", subcore_axis_name="s")

    @functools.partial(
        pl.kernel, mesh=mesh,
        out_type=jax.ShapeDtypeStruct((B, D), jnp.float32),
        scratch_types=[
            pltpu.VMEM((bpw,), jnp.int32),
            pltpu.VMEM((2 * 128, D), jnp.float32),
            pltpu.SemaphoreType.DMA,
        ],
    )
    def g(table_hbm, idx_hbm, out_hbm, idx_v, rows_v, sem):
        wid = lax.axis_index("s") * NC + lax.axis_index("c")
        base = wid * bpw
        pltpu.sync_copy(idx_hbm.at[pl.ds(base, bpw)], idx_v)
        cps = [None] * nk
        for j in range(nk):
            cps[j] = pltpu.async_copy(
                table_hbm.at[idx_v.at[pl.ds(j * 128, 128)]],
                rows_v.at[pl.ds((j % 2) * 128, 128)], sem)
            if j >= 1:
                cps[j - 1].wait()
                pltpu.sync_copy(rows_v.at[pl.ds(((j - 1) % 2) * 128, 128)],
                                out_hbm.at[pl.ds(base + (j - 1) * 128, 128)])
        cps[nk - 1].wait()
        pltpu.sync_copy(rows_v.at[pl.ds(((nk - 1) % 2) * 128, 128)],
                        out_hbm.at[pl.ds(base + (nk - 1) * 128, 128)])

    return g(table, idx)


def _nt(a, b):
    return lax.dot_general(a, b, (((1,), (1,)), ((), ())),
                           preferred_element_type=jnp.float32)


def _nn(a, b):
    return lax.dot_general(a, b, (((1,), (0,)), ((), ())),
                           preferred_element_type=jnp.float32)


def _ln(x, g, b):
    m = jnp.mean(x, axis=-1, keepdims=True)
    v = jnp.mean((x - m) ** 2, axis=-1, keepdims=True)
    return (x - m) / jnp.sqrt(v + 1e-5) * g + b


def _bn_rows(x, g, b):
    m = jnp.mean(x, axis=0, keepdims=True)
    v = jnp.mean((x - m) ** 2, axis=0, keepdims=True)
    return (x - m) / jnp.sqrt(v + 1e-5) * g + b


def _gelu(x):
    return 0.5 * x * (1.0 + lax.erf(x * (2.0 ** -0.5)))


def _dense_body(gf_ref, pos_ref,
                c1aw_ref, c1ab_ref, bn1g_ref, bn1b_ref, c1bw_ref, c1bb_ref,
                c2aw_ref, c2ab_ref, bn2g_ref, bn2b_ref, c2bw_ref, c2bb_ref,
                ln1g_ref, ln1b_ref, qkvw_ref, qkvb_ref, fcw_ref, fcb_ref,
                ln2g_ref, ln2b_ref, m1w_ref, m1b_ref, m2w_ref, m2b_ref,
                heg_ref, heb_ref, hew1_ref, heb1_ref, hew2_ref, heb2_ref,
                hpg_ref, hpb_ref, hpw1_ref, hpb1_ref, hpw2_ref, hpb2_ref,
                out_ref):
    G, M = NUM_GROUPS, GROUP_SIZE
    NM = G * M
    gf = gf_ref[...]
    h = _nt(gf, c1aw_ref[...]) + c1ab_ref[...]
    h = jnp.maximum(_bn_rows(h, bn1g_ref[...], bn1b_ref[...]), 0.0)
    h2 = _nt(h, c1bw_ref[...]) + c1bb_ref[...]
    gmax = jnp.max(h2.reshape(G, M, 64), axis=1, keepdims=True)
    gmaxb = jnp.broadcast_to(gmax, (G, M, 64)).reshape(NM, 64)
    hc = jnp.concatenate([h2, gmaxb], axis=1)
    h3 = _nt(hc, c2aw_ref[...]) + c2ab_ref[...]
    h3 = jnp.maximum(_bn_rows(h3, bn2g_ref[...], bn2b_ref[...]), 0.0)
    h4 = _nt(h3, c2bw_ref[...]) + c2bb_ref[...]
    tokens = jnp.max(h4.reshape(G, M, 128), axis=1) + pos_ref[0]

    for i in range(2):
        hh = _ln(tokens, ln1g_ref[pl.ds(i, 1), :], ln1b_ref[pl.ds(i, 1), :])
        qkv = _nt(hh, qkvw_ref[i]) + qkvb_ref[pl.ds(i, 1), :]
        heads = []
        for hd in range(8):
            q = lax.slice(qkv, (0, hd * 16), (G, hd * 16 + 16))
            k = lax.slice(qkv, (0, 128 + hd * 16), (G, 128 + hd * 16 + 16))
            v = lax.slice(qkv, (0, 256 + hd * 16), (G, 256 + hd * 16 + 16))
            s = _nt(q, k) * (16.0 ** -0.5)
            s = s - jnp.max(s, axis=-1, keepdims=True)
            e = jnp.exp(s)
            a = e / jnp.sum(e, axis=-1, keepdims=True)
            heads.append(_nn(a, v))
        ao = jnp.concatenate(heads, axis=1)
        tokens = tokens + _nt(ao, fcw_ref[i]) + fcb_ref[pl.ds(i, 1), :]
        hh = _ln(tokens, ln2g_ref[pl.ds(i, 1), :], ln2b_ref[pl.ds(i, 1), :])
        mid = _gelu(_nt(hh, m1w_ref[i]) + m1b_ref[pl.ds(i, 1), :])
        tokens = tokens + _nt(mid, m2w_ref[i]) + m2b_ref[pl.ds(i, 1), :]

    def head(x, g_, b_, w1, b1, w2, b2):
        t = _ln(x, g_, b_)
        t = _gelu(_nt(t, w1) + b1)
        return _nt(t, w2) + b2

    e = head(tokens, heg_ref[...], heb_ref[...], hew1_ref[...],
             heb1_ref[...], hew2_ref[...], heb2_ref[...])
    p = head(tokens, hpg_ref[...], hpb_ref[...], hpw1_ref[...],
             hpb1_ref[...], hpw2_ref[...], hpb2_ref[...])
    pad = jnp.zeros((NUM_GROUPS, 96), jnp.float32)
    out_ref[...] = jnp.concatenate([e, p, pad], axis=1)


def _dense(gathered, pos_emb, *ws):
    return pl.pallas_call(
        _dense_body,
        out_shape=jax.ShapeDtypeStruct((NUM_GROUPS, 128), jnp.float32),
    )(gathered, pos_emb, *ws)


def kernel(x, features, pos_emb, ge_c1a_w, ge_c1a_b, ge_bn1_g, ge_bn1_b,
           ge_c1b_w, ge_c1b_b, ge_c2a_w, ge_c2a_b, ge_bn2_g, ge_bn2_b,
           ge_c2b_w, ge_c2b_b, blk_ln1_g, blk_ln1_b, blk_qkv_w, blk_qkv_b,
           blk_fc_w, blk_fc_b, blk_ln2_g, blk_ln2_b, blk_mlp_w1, blk_mlp_b1,
           blk_mlp_w2, blk_mlp_b2, he_ln_g, he_ln_b, he_w1, he_b1, he_w2,
           he_b2, hp_ln_g, hp_ln_b, hp_w1, hp_b1, hp_w2, hp_b2):
    xpad = jnp.pad(x, ((0, NPAD - N_POINTS), (0, 0)))
    xc = xpad.T.reshape(3, NT, 128)
    xr = jnp.pad(xpad, ((0, 0), (0, 125)))

    centers = _fps(xc)
    topk, nearest = _knn(xc, xr, centers)

    idxflat = topk[:, :GROUP_SIZE].reshape(-1)
    gathered = _sc_gather(features, idxflat, 128)

    r2 = lambda a: a.reshape(1, -1)
    ws = (ge_c1a_w, r2(ge_c1a_b), r2(ge_bn1_g), r2(ge_bn1_b),
          ge_c1b_w, r2(ge_c1b_b), ge_c2a_w, r2(ge_c2a_b),
          r2(ge_bn2_g), r2(ge_bn2_b), ge_c2b_w, r2(ge_c2b_b),
          blk_ln1_g, blk_ln1_b, blk_qkv_w, blk_qkv_b, blk_fc_w, blk_fc_b,
          blk_ln2_g, blk_ln2_b, blk_mlp_w1, blk_mlp_b1, blk_mlp_w2,
          blk_mlp_b2,
          r2(he_ln_g), r2(he_ln_b), he_w1, r2(he_b1), he_w2, r2(he_b2),
          r2(hp_ln_g), r2(hp_ln_b), hp_w1, r2(hp_b1), hp_w2, r2(hp_b2))
    table = _dense(gathered, pos_emb, *ws)

    nflat = nearest.reshape(-1)
    npad2 = jnp.pad(nflat, (0, 53248 - NPAD))
    out = _sc_gather(table, npad2, 128)
    return out[:N_POINTS, :32]


# speedup vs baseline: 20.8072x; 1.2164x over previous
import functools

import jax
import jax.numpy as jnp
from jax import lax
from jax.experimental import pallas as pl
from jax.experimental.pallas import tpu as pltpu
from jax.experimental.pallas import tpu_sc as plsc

N_POINTS = 50000
NT = 392
NPAD = NT * 128
NUM_GROUPS = 512
GROUP_SIZE = 32
CB = 64
NBLK = NUM_GROUPS // CB
INF = float("inf")
IBIG = 1 << 30


def _fps_body(xc_ref, cen_ref):
    li = lax.broadcasted_iota(jnp.int32, (1, 128), 1)
    r0 = lax.broadcasted_iota(jnp.int32, (NT, 128), 0)
    l0 = lax.broadcasted_iota(jnp.int32, (NT, 128), 1)
    fid = r0 * 128 + l0
    valid = fid < N_POINTS

    def body(i, carry):
        f, dist = carry
        r = f // 128
        l = f - r * 128
        row0 = xc_ref[0, pl.ds(r, 1), :]
        row1 = xc_ref[1, pl.ds(r, 1), :]
        row2 = xc_ref[2, pl.ds(r, 1), :]
        sel = (li == l).astype(jnp.float32)
        cx = jnp.sum(row0 * sel)
        cy = jnp.sum(row1 * sel)
        cz = jnp.sum(row2 * sel)
        lif = li.astype(jnp.float32)
        rec = (jnp.where(li == 0, cx, 0.0) + jnp.where(li == 1, cy, 0.0)
               + jnp.where(li == 2, cz, 0.0))
        cen_ref[pl.ds(i, 1), :] = rec
        dx = xc_ref[0] - cx
        dy = xc_ref[1] - cy
        dz = xc_ref[2] - cz
        d = dx * dx + dy * dy + dz * dz
        dist = jnp.where(valid, jnp.minimum(dist, d), dist)
        m = jnp.max(dist)
        f_new = jnp.min(jnp.where(dist == m, fid, IBIG))
        return f_new, dist

    dist0 = jnp.where(valid, jnp.float32(1e10), jnp.float32(-1.0))
    lax.fori_loop(0, NUM_GROUPS, body, (jnp.int32(0), dist0))


def _fps(xc):
    return pl.pallas_call(
        _fps_body,
        out_shape=jax.ShapeDtypeStruct((NUM_GROUPS, 128), jnp.float32),
    )(xc)


def _knn_body(xc_ref, xr_ref, cen_ref, topk_ref, near_ref,
              *refs):
    b = pl.program_id(0)
    li = lax.broadcasted_iota(jnp.int32, (1, 128), 1)
    ci8 = lax.broadcasted_iota(jnp.int32, (CB, 1), 0)
    it392 = lax.broadcasted_iota(jnp.int32, (CB, NT), 1)
    is8 = lax.broadcasted_iota(jnp.int32, (CB, NT), 0)
    cr8 = lax.broadcasted_iota(jnp.int32, (CB, 128), 0)
    cl8 = lax.broadcasted_iota(jnp.int32, (CB, 128), 1)
    drefs = list(refs[:CB])
    xx_ref, bestv_ref, besti_ref = refs[CB], refs[CB + 1], refs[CB + 2]

    cen8 = cen_ref[...]
    cx = cen_ref[:, 0:1]
    cy = cen_ref[:, 1:2]
    cz = cen_ref[:, 2:3]
    cc = (cx * cx + cy * cy) + cz * cz

    @pl.when(b == 0)
    def _init():
        x0 = xc_ref[0]
        x1 = xc_ref[1]
        x2 = xc_ref[2]
        xx = (x0 * x0 + x1 * x1) + x2 * x2
        r0 = lax.broadcasted_iota(jnp.int32, (NT, 128), 0)
        l0 = lax.broadcasted_iota(jnp.int32, (NT, 128), 1)
        fid = r0 * 128 + l0
        xx_ref[...] = jnp.where(fid < N_POINTS, xx, jnp.float32(3e38))
        bestv_ref[...] = jnp.full((NT, 128), INF)
        besti_ref[...] = jnp.zeros((NT, 128), jnp.int32)

    def fill(s, cm):
        xs = xr_ref[pl.ds(s * 1024, 1024), :]
        cdot = lax.dot_general(cen8, xs, (((1,), (1,)), ((), ())),
                               preferred_element_type=jnp.float32)
        for k in range(8):
            t = s * 8 + k
            dk = lax.slice(cdot, (0, k * 128), (CB, (k + 1) * 128))
            xxr = xx_ref[pl.ds(t, 1), :]
            d = (-2.0 * dk + cc) + xxr
            for c in range(CB):
                drefs[c][pl.ds(t, 1), :] = lax.slice(d, (c, 0), (c + 1, 128))
            colmin = jnp.min(d, axis=1, keepdims=True)
            cm = jnp.where(it392 == t, colmin, cm)
            dmin = jnp.min(d, axis=0, keepdims=True)
            nidx = jnp.min(jnp.where(d == dmin, ci8, IBIG), axis=0,
                           keepdims=True) + CB * b
            bv = bestv_ref[pl.ds(t, 1), :]
            upd = dmin < bv
            bestv_ref[pl.ds(t, 1), :] = jnp.where(upd, dmin, bv)
            bi = besti_ref[pl.ds(t, 1), :]
            besti_ref[pl.ds(t, 1), :] = jnp.where(upd, nidx, bi)
        return cm

    cm0 = lax.fori_loop(0, NT // 8, fill, jnp.full((CB, NT), INF))

    def ext(j, carry):
        cm, res = carry
        m8 = jnp.min(cm, axis=1, keepdims=True)
        t8 = jnp.min(jnp.where(cm == m8, it392, IBIG), axis=1,
                     keepdims=True)
        for c in range(CB):
            t_c = jnp.sum(lax.slice(t8, (c, 0), (c + 1, 1)))
            row = drefs[c][pl.ds(t_c, 1), :]
            rmv = jnp.min(row, axis=1, keepdims=True)
            lv = jnp.min(jnp.where(row == rmv, li, IBIG), axis=1,
                         keepdims=True)
            new_row = jnp.where(li == lv, INF, row)
            drefs[c][pl.ds(t_c, 1), :] = new_row
            nmv = jnp.min(new_row, axis=1, keepdims=True)
            g = t_c * 128 + lv
            res = jnp.where((cr8 == c) & (cl8 == j), g, res)
            cm = jnp.where((is8 == c) & (it392 == t_c), nmv, cm)
        return cm, res

    _, res = lax.fori_loop(0, GROUP_SIZE, ext,
                           (cm0, jnp.zeros((CB, 128), jnp.int32)))
    topk_ref[...] = res
    near_ref[...] = besti_ref[...]


def _knn(xc, xr, centers):
    return pl.pallas_call(
        _knn_body,
        grid=(NBLK,),
        in_specs=[
            pl.BlockSpec((3, NT, 128), lambda b: (0, 0, 0)),
            pl.BlockSpec((NPAD, 128), lambda b: (0, 0)),
            pl.BlockSpec((CB, 128), lambda b: (b, 0)),
        ],
        out_specs=[
            pl.BlockSpec((CB, 128), lambda b: (b, 0)),
            pl.BlockSpec((NT, 128), lambda b: (0, 0)),
        ],
        out_shape=[
            jax.ShapeDtypeStruct((NUM_GROUPS, 128), jnp.int32),
            jax.ShapeDtypeStruct((NT, 128), jnp.int32),
        ],
        scratch_shapes=(
            [pltpu.VMEM((NT, 128), jnp.float32) for _ in range(CB)]
            + [pltpu.VMEM((NT, 128), jnp.float32),
               pltpu.VMEM((NT, 128), jnp.float32),
               pltpu.VMEM((NT, 128), jnp.int32)]
        ),
    )(xc, xr, centers)


def _sc_gather(table, idx, D):
    B = idx.shape[0]
    info = plsc.get_sparse_core_info()
    NC, NS = info.num_cores, info.num_subcores
    NW = NC * NS
    bpw = B // NW
    nk = bpw // 128
    mesh = plsc.VectorSubcoreMesh(core_axis_name="c", subcore_axis_name="s")

    @functools.partial(
        pl.kernel, mesh=mesh,
        out_type=jax.ShapeDtypeStruct((B, D), jnp.float32),
        scratch_types=[
            pltpu.VMEM((bpw,), jnp.int32),
            pltpu.VMEM((2 * 128, D), jnp.float32),
            pltpu.SemaphoreType.DMA,
        ],
    )
    def g(table_hbm, idx_hbm, out_hbm, idx_v, rows_v, sem):
        wid = lax.axis_index("s") * NC + lax.axis_index("c")
        base = wid * bpw
        pltpu.sync_copy(idx_hbm.at[pl.ds(base, bpw)], idx_v)
        cps = [None] * nk
        for j in range(nk):
            cps[j] = pltpu.async_copy(
                table_hbm.at[idx_v.at[pl.ds(j * 128, 128)]],
                rows_v.at[pl.ds((j % 2) * 128, 128)], sem)
            if j >= 1:
                cps[j - 1].wait()
                pltpu.sync_copy(rows_v.at[pl.ds(((j - 1) % 2) * 128, 128)],
                                out_hbm.at[pl.ds(base + (j - 1) * 128, 128)])
        cps[nk - 1].wait()
        pltpu.sync_copy(rows_v.at[pl.ds(((nk - 1) % 2) * 128, 128)],
                        out_hbm.at[pl.ds(base + (nk - 1) * 128, 128)])

    return g(table, idx)


def _nt(a, b):
    return lax.dot_general(a, b, (((1,), (1,)), ((), ())),
                           preferred_element_type=jnp.float32)


def _nn(a, b):
    return lax.dot_general(a, b, (((1,), (0,)), ((), ())),
                           preferred_element_type=jnp.float32)


def _ln(x, g, b):
    m = jnp.mean(x, axis=-1, keepdims=True)
    v = jnp.mean((x - m) ** 2, axis=-1, keepdims=True)
    return (x - m) / jnp.sqrt(v + 1e-5) * g + b


def _bn_rows(x, g, b):
    m = jnp.mean(x, axis=0, keepdims=True)
    v = jnp.mean((x - m) ** 2, axis=0, keepdims=True)
    return (x - m) / jnp.sqrt(v + 1e-5) * g + b


def _gelu(x):
    return 0.5 * x * (1.0 + lax.erf(x * (2.0 ** -0.5)))


def _dense_body(gf_ref, pos_ref,
                c1aw_ref, c1ab_ref, bn1g_ref, bn1b_ref, c1bw_ref, c1bb_ref,
                c2aw_ref, c2ab_ref, bn2g_ref, bn2b_ref, c2bw_ref, c2bb_ref,
                ln1g_ref, ln1b_ref, qkvw_ref, qkvb_ref, fcw_ref, fcb_ref,
                ln2g_ref, ln2b_ref, m1w_ref, m1b_ref, m2w_ref, m2b_ref,
                heg_ref, heb_ref, hew1_ref, heb1_ref, hew2_ref, heb2_ref,
                hpg_ref, hpb_ref, hpw1_ref, hpb1_ref, hpw2_ref, hpb2_ref,
                out_ref):
    G, M = NUM_GROUPS, GROUP_SIZE
    NM = G * M
    gf = gf_ref[...]
    h = _nt(gf, c1aw_ref[...]) + c1ab_ref[...]
    h = jnp.maximum(_bn_rows(h, bn1g_ref[...], bn1b_ref[...]), 0.0)
    h2 = _nt(h, c1bw_ref[...]) + c1bb_ref[...]
    gmax = jnp.max(h2.reshape(G, M, 64), axis=1, keepdims=True)
    gmaxb = jnp.broadcast_to(gmax, (G, M, 64)).reshape(NM, 64)
    hc = jnp.concatenate([h2, gmaxb], axis=1)
    h3 = _nt(hc, c2aw_ref[...]) + c2ab_ref[...]
    h3 = jnp.maximum(_bn_rows(h3, bn2g_ref[...], bn2b_ref[...]), 0.0)
    h4 = _nt(h3, c2bw_ref[...]) + c2bb_ref[...]
    tokens = jnp.max(h4.reshape(G, M, 128), axis=1) + pos_ref[0]

    for i in range(2):
        hh = _ln(tokens, ln1g_ref[pl.ds(i, 1), :], ln1b_ref[pl.ds(i, 1), :])
        qkv = _nt(hh, qkvw_ref[i]) + qkvb_ref[pl.ds(i, 1), :]
        heads = []
        for hd in range(8):
            q = lax.slice(qkv, (0, hd * 16), (G, hd * 16 + 16))
            k = lax.slice(qkv, (0, 128 + hd * 16), (G, 128 + hd * 16 + 16))
            v = lax.slice(qkv, (0, 256 + hd * 16), (G, 256 + hd * 16 + 16))
            s = _nt(q, k) * (16.0 ** -0.5)
            s = s - jnp.max(s, axis=-1, keepdims=True)
            e = jnp.exp(s)
            a = e / jnp.sum(e, axis=-1, keepdims=True)
            heads.append(_nn(a, v))
        ao = jnp.concatenate(heads, axis=1)
        tokens = tokens + _nt(ao, fcw_ref[i]) + fcb_ref[pl.ds(i, 1), :]
        hh = _ln(tokens, ln2g_ref[pl.ds(i, 1), :], ln2b_ref[pl.ds(i, 1), :])
        mid = _gelu(_nt(hh, m1w_ref[i]) + m1b_ref[pl.ds(i, 1), :])
        tokens = tokens + _nt(mid, m2w_ref[i]) + m2b_ref[pl.ds(i, 1), :]

    def head(x, g_, b_, w1, b1, w2, b2):
        t = _ln(x, g_, b_)
        t = _gelu(_nt(t, w1) + b1)
        return _nt(t, w2) + b2

    e = head(tokens, heg_ref[...], heb_ref[...], hew1_ref[...],
             heb1_ref[...], hew2_ref[...], heb2_ref[...])
    p = head(tokens, hpg_ref[...], hpb_ref[...], hpw1_ref[...],
             hpb1_ref[...], hpw2_ref[...], hpb2_ref[...])
    pad = jnp.zeros((NUM_GROUPS, 96), jnp.float32)
    out_ref[...] = jnp.concatenate([e, p, pad], axis=1)


def _dense(gathered, pos_emb, *ws):
    return pl.pallas_call(
        _dense_body,
        out_shape=jax.ShapeDtypeStruct((NUM_GROUPS, 128), jnp.float32),
    )(gathered, pos_emb, *ws)


def kernel(x, features, pos_emb, ge_c1a_w, ge_c1a_b, ge_bn1_g, ge_bn1_b,
           ge_c1b_w, ge_c1b_b, ge_c2a_w, ge_c2a_b, ge_bn2_g, ge_bn2_b,
           ge_c2b_w, ge_c2b_b, blk_ln1_g, blk_ln1_b, blk_qkv_w, blk_qkv_b,
           blk_fc_w, blk_fc_b, blk_ln2_g, blk_ln2_b, blk_mlp_w1, blk_mlp_b1,
           blk_mlp_w2, blk_mlp_b2, he_ln_g, he_ln_b, he_w1, he_b1, he_w2,
           he_b2, hp_ln_g, hp_ln_b, hp_w1, hp_b1, hp_w2, hp_b2):
    xpad = jnp.pad(x, ((0, NPAD - N_POINTS), (0, 0)))
    xc = xpad.T.reshape(3, NT, 128)
    xr = jnp.pad(xpad, ((0, 0), (0, 125)))

    centers = _fps(xc)
    topk, nearest = _knn(xc, xr, centers)

    idxflat = topk[:, :GROUP_SIZE].reshape(-1)
    gathered = _sc_gather(features, idxflat, 128)

    r2 = lambda a: a.reshape(1, -1)
    ws = (ge_c1a_w, r2(ge_c1a_b), r2(ge_bn1_g), r2(ge_bn1_b),
          ge_c1b_w, r2(ge_c1b_b), ge_c2a_w, r2(ge_c2a_b),
          r2(ge_bn2_g), r2(ge_bn2_b), ge_c2b_w, r2(ge_c2b_b),
          blk_ln1_g, blk_ln1_b, blk_qkv_w, blk_qkv_b, blk_fc_w, blk_fc_b,
          blk_ln2_g, blk_ln2_b, blk_mlp_w1, blk_mlp_b1, blk_mlp_w2,
          blk_mlp_b2,
          r2(he_ln_g), r2(he_ln_b), he_w1, r2(he_b1), he_w2, r2(he_b2),
          r2(hp_ln_g), r2(hp_ln_b), hp_w1, r2(hp_b1), hp_w2, r2(hp_b2))
    table = _dense(gathered, pos_emb, *ws)

    nflat = nearest.reshape(-1)
    npad2 = jnp.pad(nflat, (0, 53248 - NPAD))
    out = _sc_gather(table, npad2, 128)
    return out[:N_POINTS, :32]


# speedup vs baseline: 21.7997x; 1.0477x over previous
import functools

import jax
import jax.numpy as jnp
from jax import lax
from jax.experimental import pallas as pl
from jax.experimental.pallas import tpu as pltpu
from jax.experimental.pallas import tpu_sc as plsc

N_POINTS = 50000
NT = 392
NPAD = NT * 128
NUM_GROUPS = 512
GROUP_SIZE = 32
CB = 128
NBLK = NUM_GROUPS // CB
INF = float("inf")
IBIG = 1 << 30


def _fps_body(xc_ref, cen_ref):
    li = lax.broadcasted_iota(jnp.int32, (1, 128), 1)
    r0 = lax.broadcasted_iota(jnp.int32, (NT, 128), 0)
    l0 = lax.broadcasted_iota(jnp.int32, (NT, 128), 1)
    fid = r0 * 128 + l0
    valid = fid < N_POINTS

    def body(i, carry):
        f, dist = carry
        r = f // 128
        l = f - r * 128
        row0 = xc_ref[0, pl.ds(r, 1), :]
        row1 = xc_ref[1, pl.ds(r, 1), :]
        row2 = xc_ref[2, pl.ds(r, 1), :]
        sel = (li == l).astype(jnp.float32)
        cx = jnp.sum(row0 * sel)
        cy = jnp.sum(row1 * sel)
        cz = jnp.sum(row2 * sel)
        lif = li.astype(jnp.float32)
        rec = (jnp.where(li == 0, cx, 0.0) + jnp.where(li == 1, cy, 0.0)
               + jnp.where(li == 2, cz, 0.0))
        cen_ref[pl.ds(i, 1), :] = rec
        dx = xc_ref[0] - cx
        dy = xc_ref[1] - cy
        dz = xc_ref[2] - cz
        d = dx * dx + dy * dy + dz * dz
        dist = jnp.where(valid, jnp.minimum(dist, d), dist)
        m = jnp.max(dist)
        f_new = jnp.min(jnp.where(dist == m, fid, IBIG))
        return f_new, dist

    dist0 = jnp.where(valid, jnp.float32(1e10), jnp.float32(-1.0))
    lax.fori_loop(0, NUM_GROUPS, body, (jnp.int32(0), dist0))


def _fps(xc):
    return pl.pallas_call(
        _fps_body,
        out_shape=jax.ShapeDtypeStruct((NUM_GROUPS, 128), jnp.float32),
    )(xc)


def _knn_body(xc_ref, xr_ref, cen_ref, topk_ref, near_ref,
              *refs):
    b = pl.program_id(0)
    li = lax.broadcasted_iota(jnp.int32, (1, 128), 1)
    ci8 = lax.broadcasted_iota(jnp.int32, (CB, 1), 0)
    it392 = lax.broadcasted_iota(jnp.int32, (CB, NT), 1)
    is8 = lax.broadcasted_iota(jnp.int32, (CB, NT), 0)
    cr8 = lax.broadcasted_iota(jnp.int32, (CB, 128), 0)
    cl8 = lax.broadcasted_iota(jnp.int32, (CB, 128), 1)
    drefs = list(refs[:CB])
    xx_ref, bestv_ref, besti_ref = refs[CB], refs[CB + 1], refs[CB + 2]

    cen8 = cen_ref[...]
    cx = cen_ref[:, 0:1]
    cy = cen_ref[:, 1:2]
    cz = cen_ref[:, 2:3]
    cc = (cx * cx + cy * cy) + cz * cz

    @pl.when(b == 0)
    def _init():
        x0 = xc_ref[0]
        x1 = xc_ref[1]
        x2 = xc_ref[2]
        xx = (x0 * x0 + x1 * x1) + x2 * x2
        r0 = lax.broadcasted_iota(jnp.int32, (NT, 128), 0)
        l0 = lax.broadcasted_iota(jnp.int32, (NT, 128), 1)
        fid = r0 * 128 + l0
        xx_ref[...] = jnp.where(fid < N_POINTS, xx, jnp.float32(3e38))
        bestv_ref[...] = jnp.full((NT, 128), INF)
        besti_ref[...] = jnp.zeros((NT, 128), jnp.int32)

    def fill(s, cm):
        xs = xr_ref[pl.ds(s * 1024, 1024), :]
        cdot = lax.dot_general(cen8, xs, (((1,), (1,)), ((), ())),
                               preferred_element_type=jnp.float32)
        for k in range(8):
            t = s * 8 + k
            dk = lax.slice(cdot, (0, k * 128), (CB, (k + 1) * 128))
            xxr = xx_ref[pl.ds(t, 1), :]
            d = (-2.0 * dk + cc) + xxr
            for c in range(CB):
                drefs[c][pl.ds(t, 1), :] = lax.slice(d, (c, 0), (c + 1, 128))
            colmin = jnp.min(d, axis=1, keepdims=True)
            cm = jnp.where(it392 == t, colmin, cm)
            dmin = jnp.min(d, axis=0, keepdims=True)
            nidx = jnp.min(jnp.where(d == dmin, ci8, IBIG), axis=0,
                           keepdims=True) + CB * b
            bv = bestv_ref[pl.ds(t, 1), :]
            upd = dmin < bv
            bestv_ref[pl.ds(t, 1), :] = jnp.where(upd, dmin, bv)
            bi = besti_ref[pl.ds(t, 1), :]
            besti_ref[pl.ds(t, 1), :] = jnp.where(upd, nidx, bi)
        return cm

    cm0 = lax.fori_loop(0, NT // 8, fill, jnp.full((CB, NT), INF))

    def ext(j, carry):
        cm, res = carry
        m8 = jnp.min(cm, axis=1, keepdims=True)
        t8 = jnp.min(jnp.where(cm == m8, it392, IBIG), axis=1,
                     keepdims=True)
        for c in range(CB):
            t_c = jnp.sum(lax.slice(t8, (c, 0), (c + 1, 1)))
            row = drefs[c][pl.ds(t_c, 1), :]
            rmv = jnp.min(row, axis=1, keepdims=True)
            lv = jnp.min(jnp.where(row == rmv, li, IBIG), axis=1,
                         keepdims=True)
            new_row = jnp.where(li == lv, INF, row)
            drefs[c][pl.ds(t_c, 1), :] = new_row
            nmv = jnp.min(new_row, axis=1, keepdims=True)
            g = t_c * 128 + lv
            res = jnp.where((cr8 == c) & (cl8 == j), g, res)
            cm = jnp.where((is8 == c) & (it392 == t_c), nmv, cm)
        return cm, res

    _, res = lax.fori_loop(0, GROUP_SIZE, ext,
                           (cm0, jnp.zeros((CB, 128), jnp.int32)))
    topk_ref[...] = res
    near_ref[...] = besti_ref[...]


def _knn(xc, xr, centers):
    return pl.pallas_call(
        _knn_body,
        grid=(NBLK,),
        in_specs=[
            pl.BlockSpec((3, NT, 128), lambda b: (0, 0, 0)),
            pl.BlockSpec((NPAD, 128), lambda b: (0, 0)),
            pl.BlockSpec((CB, 128), lambda b: (b, 0)),
        ],
        out_specs=[
            pl.BlockSpec((CB, 128), lambda b: (b, 0)),
            pl.BlockSpec((NT, 128), lambda b: (0, 0)),
        ],
        out_shape=[
            jax.ShapeDtypeStruct((NUM_GROUPS, 128), jnp.int32),
            jax.ShapeDtypeStruct((NT, 128), jnp.int32),
        ],
        scratch_shapes=(
            [pltpu.VMEM((NT, 128), jnp.float32) for _ in range(CB)]
            + [pltpu.VMEM((NT, 128), jnp.float32),
               pltpu.VMEM((NT, 128), jnp.float32),
               pltpu.VMEM((NT, 128), jnp.int32)]
        ),
    )(xc, xr, centers)


def _sc_gather(table, idx, D):
    B = idx.shape[0]
    info = plsc.get_sparse_core_info()
    NC, NS = info.num_cores, info.num_subcores
    NW = NC * NS
    bpw = B // NW
    nk = bpw // 128
    mesh = plsc.VectorSubcoreMesh(core_axis_name="c", subcore_axis_name="s")

    @functools.partial(
        pl.kernel, mesh=mesh,
        out_type=jax.ShapeDtypeStruct((B, D), jnp.float32),
        scratch_types=[
            pltpu.VMEM((bpw,), jnp.int32),
            pltpu.VMEM((2 * 128, D), jnp.float32),
            pltpu.SemaphoreType.DMA,
        ],
    )
    def g(table_hbm, idx_hbm, out_hbm, idx_v, rows_v, sem):
        wid = lax.axis_index("s") * NC + lax.axis_index("c")
        base = wid * bpw
        pltpu.sync_copy(idx_hbm.at[pl.ds(base, bpw)], idx_v)
        cps = [None] * nk
        for j in range(nk):
            cps[j] = pltpu.async_copy(
                table_hbm.at[idx_v.at[pl.ds(j * 128, 128)]],
                rows_v.at[pl.ds((j % 2) * 128, 128)], sem)
            if j >= 1:
                cps[j - 1].wait()
                pltpu.sync_copy(rows_v.at[pl.ds(((j - 1) % 2) * 128, 128)],
                                out_hbm.at[pl.ds(base + (j - 1) * 128, 128)])
        cps[nk - 1].wait()
        pltpu.sync_copy(rows_v.at[pl.ds(((nk - 1) % 2) * 128, 128)],
                        out_hbm.at[pl.ds(base + (nk - 1) * 128, 128)])

    return g(table, idx)


def _nt(a, b):
    return lax.dot_general(a, b, (((1,), (1,)), ((), ())),
                           preferred_element_type=jnp.float32)


def _nn(a, b):
    return lax.dot_general(a, b, (((1,), (0,)), ((), ())),
                           preferred_element_type=jnp.float32)


def _ln(x, g, b):
    m = jnp.mean(x, axis=-1, keepdims=True)
    v = jnp.mean((x - m) ** 2, axis=-1, keepdims=True)
    return (x - m) / jnp.sqrt(v + 1e-5) * g + b


def _bn_rows(x, g, b):
    m = jnp.mean(x, axis=0, keepdims=True)
    v = jnp.mean((x - m) ** 2, axis=0, keepdims=True)
    return (x - m) / jnp.sqrt(v + 1e-5) * g + b


def _gelu(x):
    return 0.5 * x * (1.0 + lax.erf(x * (2.0 ** -0.5)))


def _dense_body(gf_ref, pos_ref,
                c1aw_ref, c1ab_ref, bn1g_ref, bn1b_ref, c1bw_ref, c1bb_ref,
                c2aw_ref, c2ab_ref, bn2g_ref, bn2b_ref, c2bw_ref, c2bb_ref,
                ln1g_ref, ln1b_ref, qkvw_ref, qkvb_ref, fcw_ref, fcb_ref,
                ln2g_ref, ln2b_ref, m1w_ref, m1b_ref, m2w_ref, m2b_ref,
                heg_ref, heb_ref, hew1_ref, heb1_ref, hew2_ref, heb2_ref,
                hpg_ref, hpb_ref, hpw1_ref, hpb1_ref, hpw2_ref, hpb2_ref,
                out_ref):
    G, M = NUM_GROUPS, GROUP_SIZE
    NM = G * M
    gf = gf_ref[...]
    h = _nt(gf, c1aw_ref[...]) + c1ab_ref[...]
    h = jnp.maximum(_bn_rows(h, bn1g_ref[...], bn1b_ref[...]), 0.0)
    h2 = _nt(h, c1bw_ref[...]) + c1bb_ref[...]
    gmax = jnp.max(h2.reshape(G, M, 64), axis=1, keepdims=True)
    gmaxb = jnp.broadcast_to(gmax, (G, M, 64)).reshape(NM, 64)
    hc = jnp.concatenate([h2, gmaxb], axis=1)
    h3 = _nt(hc, c2aw_ref[...]) + c2ab_ref[...]
    h3 = jnp.maximum(_bn_rows(h3, bn2g_ref[...], bn2b_ref[...]), 0.0)
    h4 = _nt(h3, c2bw_ref[...]) + c2bb_ref[...]
    tokens = jnp.max(h4.reshape(G, M, 128), axis=1) + pos_ref[0]

    for i in range(2):
        hh = _ln(tokens, ln1g_ref[pl.ds(i, 1), :], ln1b_ref[pl.ds(i, 1), :])
        qkv = _nt(hh, qkvw_ref[i]) + qkvb_ref[pl.ds(i, 1), :]
        heads = []
        for hd in range(8):
            q = lax.slice(qkv, (0, hd * 16), (G, hd * 16 + 16))
            k = lax.slice(qkv, (0, 128 + hd * 16), (G, 128 + hd * 16 + 16))
            v = lax.slice(qkv, (0, 256 + hd * 16), (G, 256 + hd * 16 + 16))
            s = _nt(q, k) * (16.0 ** -0.5)
            s = s - jnp.max(s, axis=-1, keepdims=True)
            e = jnp.exp(s)
            a = e / jnp.sum(e, axis=-1, keepdims=True)
            heads.append(_nn(a, v))
        ao = jnp.concatenate(heads, axis=1)
        tokens = tokens + _nt(ao, fcw_ref[i]) + fcb_ref[pl.ds(i, 1), :]
        hh = _ln(tokens, ln2g_ref[pl.ds(i, 1), :], ln2b_ref[pl.ds(i, 1), :])
        mid = _gelu(_nt(hh, m1w_ref[i]) + m1b_ref[pl.ds(i, 1), :])
        tokens = tokens + _nt(mid, m2w_ref[i]) + m2b_ref[pl.ds(i, 1), :]

    def head(x, g_, b_, w1, b1, w2, b2):
        t = _ln(x, g_, b_)
        t = _gelu(_nt(t, w1) + b1)
        return _nt(t, w2) + b2

    e = head(tokens, heg_ref[...], heb_ref[...], hew1_ref[...],
             heb1_ref[...], hew2_ref[...], heb2_ref[...])
    p = head(tokens, hpg_ref[...], hpb_ref[...], hpw1_ref[...],
             hpb1_ref[...], hpw2_ref[...], hpb2_ref[...])
    pad = jnp.zeros((NUM_GROUPS, 96), jnp.float32)
    out_ref[...] = jnp.concatenate([e, p, pad], axis=1)


def _dense(gathered, pos_emb, *ws):
    return pl.pallas_call(
        _dense_body,
        out_shape=jax.ShapeDtypeStruct((NUM_GROUPS, 128), jnp.float32),
    )(gathered, pos_emb, *ws)


def kernel(x, features, pos_emb, ge_c1a_w, ge_c1a_b, ge_bn1_g, ge_bn1_b,
           ge_c1b_w, ge_c1b_b, ge_c2a_w, ge_c2a_b, ge_bn2_g, ge_bn2_b,
           ge_c2b_w, ge_c2b_b, blk_ln1_g, blk_ln1_b, blk_qkv_w, blk_qkv_b,
           blk_fc_w, blk_fc_b, blk_ln2_g, blk_ln2_b, blk_mlp_w1, blk_mlp_b1,
           blk_mlp_w2, blk_mlp_b2, he_ln_g, he_ln_b, he_w1, he_b1, he_w2,
           he_b2, hp_ln_g, hp_ln_b, hp_w1, hp_b1, hp_w2, hp_b2):
    xpad = jnp.pad(x, ((0, NPAD - N_POINTS), (0, 0)))
    xc = xpad.T.reshape(3, NT, 128)
    xr = jnp.pad(xpad, ((0, 0), (0, 125)))

    centers = _fps(xc)
    topk, nearest = _knn(xc, xr, centers)

    idxflat = topk[:, :GROUP_SIZE].reshape(-1)
    gathered = _sc_gather(features, idxflat, 128)

    r2 = lambda a: a.reshape(1, -1)
    ws = (ge_c1a_w, r2(ge_c1a_b), r2(ge_bn1_g), r2(ge_bn1_b),
          ge_c1b_w, r2(ge_c1b_b), ge_c2a_w, r2(ge_c2a_b),
          r2(ge_bn2_g), r2(ge_bn2_b), ge_c2b_w, r2(ge_c2b_b),
          blk_ln1_g, blk_ln1_b, blk_qkv_w, blk_qkv_b, blk_fc_w, blk_fc_b,
          blk_ln2_g, blk_ln2_b, blk_mlp_w1, blk_mlp_b1, blk_mlp_w2,
          blk_mlp_b2,
          r2(he_ln_g), r2(he_ln_b), he_w1, r2(he_b1), he_w2, r2(he_b2),
          r2(hp_ln_g), r2(hp_ln_b), hp_w1, r2(hp_b1), hp_w2, r2(hp_b2))
    table = _dense(gathered, pos_emb, *ws)

    nflat = nearest.reshape(-1)
    npad2 = jnp.pad(nflat, (0, 53248 - NPAD))
    out = _sc_gather(table, npad2, 128)
    return out[:N_POINTS, :32]
